# baseline (speedup 1.0000x reference)
import jax
import jax.numpy as jnp
from jax import lax
from jax.experimental import pallas as pl

B, T, N, D_IN = 4, 24, 8192, 32
HID, NH, P, D_OUT = 16, 4, 6, 32
G = T // P
F = NH * HID
AC = NH * P * G * G
Q = 4
NQ = N // Q
NBQ = 256
FQ = Q * F
EQ = Q * NH
DQ = Q * D_IN


def _tb_kernel(x_ref, w2dT_ref, wsrcT_ref, wdstT_ref, woutT_ref, bout_ref,
               rep_ref, out_ref, attn_ref):
    x = x_ref[0]
    x2 = x.reshape(T * NBQ, DQ)

    def matT(w, v):
        return lax.dot_general(w, v, (((1,), (1,)), ((), ())),
                               preferred_element_type=jnp.float32)

    hT = matT(w2dT_ref[...], x2)
    esT = matT(wsrcT_ref[...], x2)
    edT = matT(wdstT_ref[...], x2)
    rep = rep_ref[...]

    a_chunks = []
    o_chunks = []
    for p in range(P):
        base = p * G * NBQ
        src = [esT[:, base + i * NBQ: base + (i + 1) * NBQ] for i in range(G)]
        dst = [edT[:, base + j * NBQ: base + (j + 1) * NBQ] for j in range(G)]
        for i in range(G):
            e_row = []
            for j in range(G):
                e = src[i] + dst[j]
                e_row.append(jnp.where(e >= 0.0, e, 0.2 * e))
            m = jnp.maximum(jnp.maximum(e_row[0], e_row[1]),
                            jnp.maximum(e_row[2], e_row[3]))
            ex = [jnp.exp(e - m) for e in e_row]
            inv = 1.0 / (ex[0] + ex[1] + ex[2] + ex[3])
            a_chunks.append(jnp.concatenate([exj * inv for exj in ex], axis=1))
        a_p = jnp.concatenate(a_chunks[-G:], axis=1)
        arep = jnp.dot(rep, a_p, preferred_element_type=jnp.float32)
        hslab = hT[:, base:base + G * NBQ]
        for i in range(G):
            c = arep[:, i * G * NBQ:(i + 1) * G * NBQ] * hslab
            o_chunks.append(c[:, 0:NBQ] + c[:, NBQ:2 * NBQ]
                            + c[:, 2 * NBQ:3 * NBQ] + c[:, 3 * NBQ:4 * NBQ])

    oT = jnp.concatenate(o_chunks, axis=1)
    zT = jnp.dot(woutT_ref[...], oT, preferred_element_type=jnp.float32)
    zT = zT + bout_ref[...]
    zT = jnp.where(zT > 0.0, zT, jnp.exp(zT) - 1.0)
    out_ref[0] = (x2 + zT.T).reshape(T, NBQ, DQ)

    a_cat = jnp.concatenate(a_chunks, axis=1)
    attn_ref[...] = a_cat.reshape(EQ * P * G * G, NBQ).T


def kernel(input, covariate, W, a_src, a_dst, W_out, b_out):
    del covariate
    eyeq = jnp.eye(Q, dtype=jnp.float32)
    w2dT = jnp.transpose(W, (0, 2, 1)).reshape(F, D_IN)
    w2dT4 = jnp.kron(eyeq, w2dT)
    wsrcT4 = jnp.kron(eyeq, jnp.einsum('ndh,nh->nd', W, a_src))
    wdstT4 = jnp.kron(eyeq, jnp.einsum('ndh,nh->nd', W, a_dst))
    woutT4 = jnp.kron(eyeq, W_out.T)
    bout4 = jnp.tile(b_out, Q).reshape(DQ, 1)
    rep = jnp.kron(eyeq, (jnp.arange(F)[:, None] // HID
                          == jnp.arange(NH)[None, :]).astype(jnp.float32))

    x4 = input.reshape(B, T, NQ, DQ)
    nblk = NQ // NBQ
    out, attn2 = pl.pallas_call(
        _tb_kernel,
        grid=(B, nblk),
        in_specs=[
            pl.BlockSpec((1, T, NBQ, DQ), lambda b, k: (b, 0, k, 0)),
            pl.BlockSpec((FQ, DQ), lambda b, k: (0, 0)),
            pl.BlockSpec((EQ, DQ), lambda b, k: (0, 0)),
            pl.BlockSpec((EQ, DQ), lambda b, k: (0, 0)),
            pl.BlockSpec((DQ, FQ), lambda b, k: (0, 0)),
            pl.BlockSpec((DQ, 1), lambda b, k: (0, 0)),
            pl.BlockSpec((FQ, EQ), lambda b, k: (0, 0)),
        ],
        out_specs=[
            pl.BlockSpec((1, T, NBQ, DQ), lambda b, k: (b, 0, k, 0)),
            pl.BlockSpec((NBQ, Q * AC), lambda b, k: (b * nblk + k, 0)),
        ],
        out_shape=[
            jax.ShapeDtypeStruct((B, T, NQ, DQ), jnp.float32),
            jax.ShapeDtypeStruct((B * NQ, Q * AC), jnp.float32),
        ],
    )(x4, w2dT4, wsrcT4, wdstT4, woutT4, bout4, rep)

    return (out.reshape(B, T, N, D_IN), attn2.reshape(B * N, NH, P, G, G))


# speedup vs baseline: 4.5453x; 4.5453x over previous
import jax
import jax.numpy as jnp
from jax.experimental import pallas as pl

B, T, N, D_IN = 4, 24, 8192, 32
HID, NH, P, D_OUT = 16, 4, 6, 32
G = T // P
F = NH * HID
NB = 256
AC = NH * P * G * G


def _tb_kernel(x_ref, w2dT_ref, wsrcT_ref, wdstT_ref, woutT_ref, bout_ref,
               rep_ref, perm_ref, out_ref, attn_ref):
    x = x_ref[0]
    xT = x.reshape(T * NB, D_IN).T

    hT = jnp.dot(w2dT_ref[...], xT, preferred_element_type=jnp.float32)
    esT = jnp.dot(wsrcT_ref[...], xT, preferred_element_type=jnp.float32)
    edT = jnp.dot(wdstT_ref[...], xT, preferred_element_type=jnp.float32)
    rep = rep_ref[...]

    o_chunks = []
    a_pieces = []
    for p in range(P):
        base = p * G * NB
        hslab = hT[:, base:base + G * NB]
        src = [esT[:, base + i * NB: base + (i + 1) * NB] for i in range(G)]
        dst = [edT[:, base + j * NB: base + (j + 1) * NB] for j in range(G)]
        for i in range(G):
            e_row = []
            for j in range(G):
                e = src[i] + dst[j]
                e_row.append(jnp.where(e >= 0.0, e, 0.2 * e))
            m = jnp.maximum(jnp.maximum(e_row[0], e_row[1]),
                            jnp.maximum(e_row[2], e_row[3]))
            ex = [jnp.exp(e - m) for e in e_row]
            inv = 1.0 / (ex[0] + ex[1] + ex[2] + ex[3])
            a_row = [exj * inv for exj in ex]
            a_pieces.extend(a_row)
            a_pi = jnp.concatenate(a_row, axis=1)
            arep = jnp.dot(rep, a_pi, preferred_element_type=jnp.float32)
            c = arep * hslab
            o_chunks.append(c[:, 0:NB] + c[:, NB:2 * NB]
                            + c[:, 2 * NB:3 * NB] + c[:, 3 * NB:4 * NB])

    oT = jnp.concatenate(o_chunks, axis=1)
    zT = jnp.dot(woutT_ref[...], oT, preferred_element_type=jnp.float32)
    zT = zT + bout_ref[...]
    zT = jnp.where(zT > 0.0, zT, jnp.exp(zT) - 1.0)
    outT = xT + zT
    out_ref[0] = outT.T.reshape(T, NB, D_IN)

    a0 = jnp.concatenate(a_pieces, axis=0)
    attn_ref[...] = jnp.dot(a0.T, perm_ref[...],
                            preferred_element_type=jnp.float32)


def kernel(input, covariate, W, a_src, a_dst, W_out, b_out):
    del covariate
    w2dT = jnp.transpose(W, (0, 2, 1)).reshape(F, D_IN)
    wsrcT = jnp.einsum('ndh,nh->nd', W, a_src)
    wdstT = jnp.einsum('ndh,nh->nd', W, a_dst)
    woutT = W_out.T
    bout = b_out.reshape(D_OUT, 1)
    rep = (jnp.arange(F)[:, None] // HID
           == jnp.arange(NH)[None, :]).astype(jnp.float32)
    d = jnp.arange(AC)
    s_of_d = (d % (P * G * G)) * NH + d // (P * G * G)
    perm = (jnp.arange(AC)[:, None] == s_of_d[None, :]).astype(jnp.float32)

    nblk = N // NB
    out, attn2 = pl.pallas_call(
        _tb_kernel,
        grid=(B, nblk),
        in_specs=[
            pl.BlockSpec((1, T, NB, D_IN), lambda b, k: (b, 0, k, 0)),
            pl.BlockSpec((F, D_IN), lambda b, k: (0, 0)),
            pl.BlockSpec((NH, D_IN), lambda b, k: (0, 0)),
            pl.BlockSpec((NH, D_IN), lambda b, k: (0, 0)),
            pl.BlockSpec((D_OUT, F), lambda b, k: (0, 0)),
            pl.BlockSpec((D_OUT, 1), lambda b, k: (0, 0)),
            pl.BlockSpec((F, NH), lambda b, k: (0, 0)),
            pl.BlockSpec((AC, AC), lambda b, k: (0, 0)),
        ],
        out_specs=[
            pl.BlockSpec((1, T, NB, D_IN), lambda b, k: (b, 0, k, 0)),
            pl.BlockSpec((NB, AC), lambda b, k: (b * nblk + k, 0)),
        ],
        out_shape=[
            jax.ShapeDtypeStruct((B, T, N, D_IN), jnp.float32),
            jax.ShapeDtypeStruct((B * N, AC), jnp.float32),
        ],
    )(input, w2dT, wsrcT, wdstT, woutT, bout, rep, perm)

    return (out, attn2.reshape(B * N, NH, P, G, G))


# speedup vs baseline: 4.7232x; 1.0391x over previous
import jax
import jax.numpy as jnp
from jax.experimental import pallas as pl

B, T, N, D_IN = 4, 24, 8192, 32
HID, NH, P, D_OUT = 16, 4, 6, 32
G = T // P
F = NH * HID
NB = 512
AC = NH * P * G * G


def _tb_kernel(x_ref, w2dT_ref, wsrcT_ref, wdstT_ref, woutT_ref, bout_ref,
               rep_ref, perm_ref, out_ref, attn_ref):
    x = x_ref[0]
    xT = x.reshape(T * NB, D_IN).T

    hT = jnp.dot(w2dT_ref[...], xT, preferred_element_type=jnp.float32)
    esT = jnp.dot(wsrcT_ref[...], xT, preferred_element_type=jnp.float32)
    edT = jnp.dot(wdstT_ref[...], xT, preferred_element_type=jnp.float32)
    rep = rep_ref[...]

    o_chunks = []
    a_pieces = []
    for p in range(P):
        base = p * G * NB
        hslab = hT[:, base:base + G * NB]
        src = [esT[:, base + i * NB: base + (i + 1) * NB] for i in range(G)]
        dst = [edT[:, base + j * NB: base + (j + 1) * NB] for j in range(G)]
        for i in range(G):
            e_row = []
            for j in range(G):
                e = src[i] + dst[j]
                e_row.append(jnp.where(e >= 0.0, e, 0.2 * e))
            m = jnp.maximum(jnp.maximum(e_row[0], e_row[1]),
                            jnp.maximum(e_row[2], e_row[3]))
            ex = [jnp.exp(e - m) for e in e_row]
            inv = 1.0 / (ex[0] + ex[1] + ex[2] + ex[3])
            a_row = [exj * inv for exj in ex]
            a_pieces.extend(a_row)
            a_pi = jnp.concatenate(a_row, axis=1)
            arep = jnp.dot(rep, a_pi, preferred_element_type=jnp.float32)
            c = arep * hslab
            o_chunks.append(c[:, 0:NB] + c[:, NB:2 * NB]
                            + c[:, 2 * NB:3 * NB] + c[:, 3 * NB:4 * NB])

    oT = jnp.concatenate(o_chunks, axis=1)
    zT = jnp.dot(woutT_ref[...], oT, preferred_element_type=jnp.float32)
    zT = zT + bout_ref[...]
    zT = jnp.where(zT > 0.0, zT, jnp.exp(zT) - 1.0)
    outT = xT + zT
    out_ref[0] = outT.T.reshape(T, NB, D_IN)

    a0 = jnp.concatenate(a_pieces, axis=0)
    attn_ref[...] = jnp.dot(a0.T, perm_ref[...],
                            preferred_element_type=jnp.float32)


def kernel(input, covariate, W, a_src, a_dst, W_out, b_out):
    del covariate
    w2dT = jnp.transpose(W, (0, 2, 1)).reshape(F, D_IN)
    wsrcT = jnp.einsum('ndh,nh->nd', W, a_src)
    wdstT = jnp.einsum('ndh,nh->nd', W, a_dst)
    woutT = W_out.T
    bout = b_out.reshape(D_OUT, 1)
    rep = (jnp.arange(F)[:, None] // HID
           == jnp.arange(NH)[None, :]).astype(jnp.float32)
    d = jnp.arange(AC)
    s_of_d = (d % (P * G * G)) * NH + d // (P * G * G)
    perm = (jnp.arange(AC)[:, None] == s_of_d[None, :]).astype(jnp.float32)

    nblk = N // NB
    out, attn2 = pl.pallas_call(
        _tb_kernel,
        grid=(B, nblk),
        in_specs=[
            pl.BlockSpec((1, T, NB, D_IN), lambda b, k: (b, 0, k, 0)),
            pl.BlockSpec((F, D_IN), lambda b, k: (0, 0)),
            pl.BlockSpec((NH, D_IN), lambda b, k: (0, 0)),
            pl.BlockSpec((NH, D_IN), lambda b, k: (0, 0)),
            pl.BlockSpec((D_OUT, F), lambda b, k: (0, 0)),
            pl.BlockSpec((D_OUT, 1), lambda b, k: (0, 0)),
            pl.BlockSpec((F, NH), lambda b, k: (0, 0)),
            pl.BlockSpec((AC, AC), lambda b, k: (0, 0)),
        ],
        out_specs=[
            pl.BlockSpec((1, T, NB, D_IN), lambda b, k: (b, 0, k, 0)),
            pl.BlockSpec((NB, AC), lambda b, k: (b * nblk + k, 0)),
        ],
        out_shape=[
            jax.ShapeDtypeStruct((B, T, N, D_IN), jnp.float32),
            jax.ShapeDtypeStruct((B * N, AC), jnp.float32),
        ],
    )(input, w2dT, wsrcT, wdstT, woutT, bout, rep, perm)

    return (out, attn2.reshape(B * N, NH, P, G, G))
